# baseline (speedup 1.0000x reference)
import functools

import jax
import jax.numpy as jnp
from jax import lax
from jax.experimental import pallas as pl
from jax.experimental.pallas import tpu as pltpu
from jax.experimental.pallas import tpu_sc as plsc


def _gelu(x):
    return 0.5 * x * (1.0 + jax.lax.erf(x * 0.7071067811865476))


def _router_body(tok_ref, alt_ref, gamma_ref, beta_ref,
                 w1x_ref, w1a_ref, bg1_ref, wg2_ref, bg2_ref,
                 xn_ref, rk_ref, ei_ref, g_ref, cnt_ref, lb_ref,
                 run_scr, fp_scr, *, TBA, NBA, E, N):
    b = pl.program_id(0)

    @pl.when(b == 0)
    def _init():
        run_scr[...] = jnp.zeros_like(run_scr)
        fp_scr[...] = jnp.zeros_like(fp_scr)

    tok = tok_ref[...]
    mu = jnp.mean(tok, axis=1, keepdims=True)
    xc = tok - mu
    var = jnp.mean(xc * xc, axis=1, keepdims=True)
    xn = xc * jax.lax.rsqrt(var + 1e-5) * gamma_ref[...] + beta_ref[...]
    xn_ref[...] = xn

    h = jax.lax.dot_general(xn, w1x_ref[...], (((1,), (1,)), ((), ())),
                            preferred_element_type=jnp.float32)
    ha = jax.lax.dot_general(alt_ref[...], w1a_ref[...], (((1,), (1,)), ((), ())),
                             preferred_element_type=jnp.float32)
    h = _gelu(h + ha + bg1_ref[...])
    logits = jax.lax.dot_general(h, wg2_ref[...], (((1,), (1,)), ((), ())),
                                 preferred_element_type=jnp.float32)
    logits = logits + bg2_ref[...]

    iota_e = jax.lax.broadcasted_iota(jnp.int32, logits.shape, 1)
    m1 = jnp.max(logits, axis=1, keepdims=True)
    i1 = jnp.min(jnp.where(logits == m1, iota_e, E), axis=1, keepdims=True)
    l2 = jnp.where(iota_e == i1, jnp.float32(-jnp.inf), logits)
    m2 = jnp.max(l2, axis=1, keepdims=True)
    i2 = jnp.min(jnp.where(l2 == m2, iota_e, E), axis=1, keepdims=True)
    r = jnp.exp(m2 - m1)
    den = 1.0 + r
    g_ref[...] = jnp.concatenate([1.0 / den, r / den], axis=1)
    ei_ref[...] = jnp.concatenate([i1, i2], axis=1)

    a1 = (iota_e == i1).astype(jnp.float32)
    a2 = (iota_e == i2).astype(jnp.float32)
    ri = jax.lax.broadcasted_iota(jnp.int32, (TBA, TBA), 0)
    ci = jax.lax.broadcasted_iota(jnp.int32, (TBA, TBA), 1)
    tri = (ri > ci).astype(jnp.float32)
    cum1 = jax.lax.dot_general(tri, a1, (((1,), (0,)), ((), ())),
                               preferred_element_type=jnp.float32)
    cum2 = jax.lax.dot_general(tri, a2, (((1,), (0,)), ((), ())),
                               preferred_element_type=jnp.float32)
    base = run_scr[...]
    cs1 = jnp.sum(a1, axis=0, keepdims=True)
    cs2 = jnp.sum(a2, axis=0, keepdims=True)
    r1 = jnp.sum(a1 * (base + cum1), axis=1, keepdims=True)
    r2 = jnp.sum(a2 * (base + cs1 + cum2), axis=1, keepdims=True)
    rk_ref[...] = jnp.concatenate([r1, r2], axis=1).astype(jnp.int32)
    run_scr[...] = base + cs1 + cs2
    cnt_ref[...] = run_scr[...]

    p = jnp.exp(logits - m1)
    p = p / jnp.sum(p, axis=1, keepdims=True)
    fp_scr[0:1, :] += cs1
    fp_scr[1:2, :] += jnp.sum(p, axis=0, keepdims=True)

    @pl.when(b == NBA - 1)
    def _fin():
        lb = (E / (N * N)) * jnp.sum(fp_scr[0:1, :] * fp_scr[1:2, :])
        lb_ref[...] = jnp.full(lb_ref.shape, lb, jnp.float32)


def _ffn_body(smap_ref, xs_ref, W1_ref, b1_ref, W2_ref, b2_ref, ys_ref, *, NBF):
    j = pl.program_id(0)
    nact = smap_ref[NBF]

    @pl.when(j < nact)
    def _():
        xb = xs_ref[...].astype(jnp.bfloat16)
        h1 = jax.lax.dot_general(xb, W1_ref[0], (((1,), (1,)), ((), ())),
                                 preferred_element_type=jnp.float32)
        h1 = _gelu(h1 + b1_ref[0]).astype(jnp.bfloat16)
        eo = jax.lax.dot_general(h1, W2_ref[0], (((1,), (1,)), ((), ())),
                                 preferred_element_type=jnp.float32)
        ys_ref[...] = eo + b2_ref[0]


def _make_scatter(N, D, NPAD, NW):
    rows_per_w = 2 * N // NW
    mesh = plsc.VectorSubcoreMesh(core_axis_name="c", subcore_axis_name="s")

    @functools.partial(
        pl.kernel, mesh=mesh,
        out_type=jax.ShapeDtypeStruct((NPAD, D), jnp.float32),
        scratch_types=[
            pltpu.VMEM((rows_per_w,), jnp.int32),
            pltpu.VMEM((rows_per_w, D), jnp.float32),
            pltpu.SemaphoreType.DMA,
        ],
    )
    def _scatter(xn_hbm, pos_hbm, xs_hbm, idx_v, rows_v, sem):
        wid = lax.axis_index("s") * 2 + lax.axis_index("c")
        abase = wid * rows_per_w
        tbase = abase % N
        pltpu.sync_copy(pos_hbm.at[pl.ds(abase, rows_per_w)], idx_v)
        pltpu.sync_copy(xn_hbm.at[pl.ds(tbase, rows_per_w)], rows_v)
        pltpu.async_copy(rows_v, xs_hbm.at[idx_v], sem).wait()

    return _scatter


def _make_combine(N, D, NW):
    CH = 32
    nch = N // (NW * CH)
    mesh = plsc.VectorSubcoreMesh(core_axis_name="c", subcore_axis_name="s")

    @functools.partial(
        pl.kernel, mesh=mesh,
        out_type=jax.ShapeDtypeStruct((N, D), jnp.float32),
        scratch_types=[
            pltpu.VMEM((CH, D), jnp.float32),
            pltpu.VMEM((CH, D), jnp.float32),
            pltpu.VMEM((CH, D), jnp.float32),
            pltpu.VMEM((CH,), jnp.int32),
            pltpu.VMEM((CH,), jnp.int32),
            pltpu.VMEM((CH * 16,), jnp.float32),
            pltpu.VMEM((CH * 16,), jnp.float32),
            pltpu.SemaphoreType.DMA,
            pltpu.SemaphoreType.DMA,
            pltpu.SemaphoreType.DMA,
        ],
    )
    def _combine(tok_hbm, ys_hbm, pos_hbm, g_hbm, out_hbm,
                 b0, b1, b2, i1v, i2v, g1v, g2v, s0, s1, s2):
        wid = lax.axis_index("s") * 2 + lax.axis_index("c")
        for c in range(nch):
            tb = wid * (nch * CH) + c * CH
            pltpu.sync_copy(pos_hbm.at[pl.ds(tb, CH)], i1v)
            pltpu.sync_copy(pos_hbm.at[pl.ds(N + tb, CH)], i2v)
            pltpu.sync_copy(g_hbm.at[pl.ds(tb * 16, CH * 16)], g1v)
            pltpu.sync_copy(g_hbm.at[pl.ds((N + tb) * 16, CH * 16)], g2v)
            cp0 = pltpu.async_copy(tok_hbm.at[pl.ds(tb, CH)], b0, s0)
            cp1 = pltpu.async_copy(ys_hbm.at[i1v], b1, s1)
            cp2 = pltpu.async_copy(ys_hbm.at[i2v], b2, s2)
            cp0.wait()
            cp1.wait()
            cp2.wait()

            def tokbody(t, _):
                g1 = g1v[pl.ds(t * 16, 16)]
                g2 = g2v[pl.ds(t * 16, 16)]

                def vbody(j, _):
                    sl = pl.ds(j * 16, 16)
                    b0[t, sl] = b0[t, sl] + g1 * b1[t, sl] + g2 * b2[t, sl]
                    return 0

                return lax.fori_loop(0, D // 16, vbody, 0)

            lax.fori_loop(0, CH, tokbody, 0)
            pltpu.sync_copy(b0, out_hbm.at[pl.ds(tb, CH)])

    return _combine


def kernel(tokens, alt_embedding, gamma, beta, Wg1, bg1, Wg2, bg2, W1, b1, W2, b2):
    B, N, D = tokens.shape
    ALT = alt_embedding.shape[-1]
    E, DFF, _ = W1.shape
    DG = Wg1.shape[0]
    TBA = 512
    NBA = N // TBA
    TBF = 256
    NBF = (2 * N) // TBF + (E - 1)
    NPAD = NBF * TBF
    NW = 32

    x = tokens.reshape(N, D)
    w1x = Wg1[:, :D]
    w1a = Wg1[:, D:]
    W1b = W1.astype(jnp.bfloat16)
    W2b = W2.astype(jnp.bfloat16)

    bodyA = functools.partial(_router_body, TBA=TBA, NBA=NBA, E=E, N=N)
    full = lambda s: pl.BlockSpec(s, lambda b: (0,) * len(s))
    xn, rks, eix, gts, cnts, lbv = pl.pallas_call(
        bodyA,
        grid=(NBA,),
        in_specs=[
            pl.BlockSpec((TBA, D), lambda b: (b, 0)),
            full((1, ALT)),
            full((1, D)),
            full((1, D)),
            full((DG, D)),
            full((DG, ALT)),
            full((1, DG)),
            full((E, DG)),
            full((1, E)),
        ],
        out_specs=[
            pl.BlockSpec((TBA, D), lambda b: (b, 0)),
            pl.BlockSpec((TBA, 2), lambda b: (b, 0)),
            pl.BlockSpec((TBA, 2), lambda b: (b, 0)),
            pl.BlockSpec((TBA, 2), lambda b: (b, 0)),
            full((1, E)),
            full((1, 8)),
        ],
        out_shape=[
            jax.ShapeDtypeStruct((N, D), jnp.float32),
            jax.ShapeDtypeStruct((N, 2), jnp.int32),
            jax.ShapeDtypeStruct((N, 2), jnp.int32),
            jax.ShapeDtypeStruct((N, 2), jnp.float32),
            jax.ShapeDtypeStruct((1, E), jnp.float32),
            jax.ShapeDtypeStruct((1, 8), jnp.float32),
        ],
        scratch_shapes=[
            pltpu.VMEM((1, E), jnp.float32),
            pltpu.VMEM((2, E), jnp.float32),
        ],
        compiler_params=pltpu.CompilerParams(
            dimension_semantics=("arbitrary",),
        ),
    )(x, alt_embedding, gamma.reshape(1, D), beta.reshape(1, D),
      w1x, w1a, bg1.reshape(1, DG), Wg2, bg2.reshape(1, E))

    cnt = cnts[0].astype(jnp.int32)
    pb = (cnt + TBF - 1) // TBF
    cum = jnp.cumsum(pb)
    po = (cum - pb) * TBF
    nact = cum[E - 1]
    jarr = jnp.arange(NBF, dtype=jnp.int32)
    bemap = jnp.minimum(jnp.sum((jarr[:, None] >= cum[None, :]).astype(jnp.int32),
                                axis=1), E - 1)
    smap = jnp.concatenate([bemap, nact[None]]).astype(jnp.int32)
    pos = jnp.take(po, eix, axis=0) + rks
    posc = jnp.concatenate([pos[:, 0], pos[:, 1]])
    gc = jnp.repeat(jnp.concatenate([gts[:, 0], gts[:, 1]]), 16)

    xs = _make_scatter(N, D, NPAD, NW)(xn, posc)

    bodyC = functools.partial(_ffn_body, NBF=NBF)
    ys = pl.pallas_call(
        bodyC,
        grid_spec=pltpu.PrefetchScalarGridSpec(
            num_scalar_prefetch=1,
            grid=(NBF,),
            in_specs=[
                pl.BlockSpec((TBF, D), lambda j, sm: (j, 0)),
                pl.BlockSpec((1, DFF, D), lambda j, sm: (sm[j], 0, 0)),
                pl.BlockSpec((1, 1, DFF), lambda j, sm: (sm[j], 0, 0)),
                pl.BlockSpec((1, D, DFF), lambda j, sm: (sm[j], 0, 0)),
                pl.BlockSpec((1, 1, D), lambda j, sm: (sm[j], 0, 0)),
            ],
            out_specs=pl.BlockSpec((TBF, D), lambda j, sm: (j, 0)),
        ),
        out_shape=jax.ShapeDtypeStruct((NPAD, D), jnp.float32),
        compiler_params=pltpu.CompilerParams(
            dimension_semantics=("arbitrary",),
        ),
    )(smap, xs, W1b, b1.reshape(E, 1, DFF), W2b, b2.reshape(E, 1, D))

    out = _make_combine(N, D, NW)(x, ys, posc, gc)
    return out.reshape(B, N, D), lbv[0, 0]


# speedup vs baseline: 1.0341x; 1.0341x over previous
import functools

import jax
import jax.numpy as jnp
from jax import lax
from jax.experimental import pallas as pl
from jax.experimental.pallas import tpu as pltpu
from jax.experimental.pallas import tpu_sc as plsc


def _gelu(x):
    return 0.5 * x * (1.0 + jax.lax.erf(x * 0.7071067811865476))


def _router_body(tok_ref, alt_ref, gamma_ref, beta_ref,
                 w1x_ref, w1a_ref, bg1_ref, wg2_ref, bg2_ref,
                 xn_ref, g_ref, pos_ref, smap_ref, lb_ref,
                 run_scr, fp_scr, ei_scr, rk_scr, *, TBA, NBA, E, N, TBF, NBF):
    b = pl.program_id(0)

    @pl.when(b == 0)
    def _init():
        run_scr[...] = jnp.zeros_like(run_scr)
        fp_scr[...] = jnp.zeros_like(fp_scr)

    tok = tok_ref[...]
    mu = jnp.mean(tok, axis=1, keepdims=True)
    xc = tok - mu
    var = jnp.mean(xc * xc, axis=1, keepdims=True)
    xn = xc * jax.lax.rsqrt(var + 1e-5) * gamma_ref[...] + beta_ref[...]
    xn_ref[...] = xn

    h = jax.lax.dot_general(xn, w1x_ref[...], (((1,), (1,)), ((), ())),
                            preferred_element_type=jnp.float32)
    ha = jax.lax.dot_general(alt_ref[...], w1a_ref[...], (((1,), (1,)), ((), ())),
                             preferred_element_type=jnp.float32)
    h = _gelu(h + ha + bg1_ref[...])
    logits = jax.lax.dot_general(h, wg2_ref[...], (((1,), (1,)), ((), ())),
                                 preferred_element_type=jnp.float32)
    logits = logits + bg2_ref[...]

    iota_e = jax.lax.broadcasted_iota(jnp.int32, logits.shape, 1)
    m1 = jnp.max(logits, axis=1, keepdims=True)
    i1 = jnp.min(jnp.where(logits == m1, iota_e, E), axis=1, keepdims=True)
    l2 = jnp.where(iota_e == i1, jnp.float32(-jnp.inf), logits)
    m2 = jnp.max(l2, axis=1, keepdims=True)
    i2 = jnp.min(jnp.where(l2 == m2, iota_e, E), axis=1, keepdims=True)
    r = jnp.exp(m2 - m1)
    den = 1.0 + r
    g_ref[...] = jnp.concatenate([1.0 / den, r / den], axis=1)

    a1 = (iota_e == i1).astype(jnp.float32)
    a2 = (iota_e == i2).astype(jnp.float32)
    ri = jax.lax.broadcasted_iota(jnp.int32, (TBA, TBA), 0)
    ci = jax.lax.broadcasted_iota(jnp.int32, (TBA, TBA), 1)
    tri = (ri > ci).astype(jnp.float32)
    cum1 = jax.lax.dot_general(tri, a1, (((1,), (0,)), ((), ())),
                               preferred_element_type=jnp.float32)
    cum2 = jax.lax.dot_general(tri, a2, (((1,), (0,)), ((), ())),
                               preferred_element_type=jnp.float32)
    base = run_scr[...]
    cs1 = jnp.sum(a1, axis=0, keepdims=True)
    cs2 = jnp.sum(a2, axis=0, keepdims=True)
    r1 = jnp.sum(a1 * (base + cum1), axis=1, keepdims=True)
    r2 = jnp.sum(a2 * (base + cs1 + cum2), axis=1, keepdims=True)
    ei_scr[pl.ds(b * TBA, TBA), :] = jnp.concatenate([i1, i2], axis=1)
    rk_scr[pl.ds(b * TBA, TBA), :] = jnp.concatenate([r1, r2], axis=1)
    run_scr[...] = base + cs1 + cs2

    p = jnp.exp(logits - m1)
    p = p / jnp.sum(p, axis=1, keepdims=True)
    fp_scr[0:1, :] += cs1
    fp_scr[1:2, :] += jnp.sum(p, axis=0, keepdims=True)

    @pl.when(b == NBA - 1)
    def _fin():
        lb = (E / (N * N)) * jnp.sum(fp_scr[0:1, :] * fp_scr[1:2, :])
        lb_ref[...] = jnp.full(lb_ref.shape, lb, jnp.float32)

        cnt = run_scr[...]
        pb_ = jnp.floor((cnt + (TBF - 1)) * (1.0 / TBF))
        li = jax.lax.broadcasted_iota(jnp.int32, (E, E), 0)
        lj = jax.lax.broadcasted_iota(jnp.int32, (E, E), 1)
        lower = (li <= lj).astype(jnp.float32)
        cum = jax.lax.dot_general(pb_, lower, (((1,), (0,)), ((), ())),
                                  preferred_element_type=jnp.float32)
        po = (cum - pb_) * TBF
        nact = cum[0:1, E - 1:E]
        jarr = jax.lax.broadcasted_iota(jnp.int32, smap_ref.shape, 1
                                        ).astype(jnp.float32)
        bemap = jnp.zeros(smap_ref.shape, jnp.float32)
        for k in range(E):
            bemap = bemap + (jarr >= cum[0:1, k:k + 1]).astype(jnp.float32)
        bemap = jnp.minimum(bemap, E - 1)
        smap_ref[...] = jnp.where(jarr == NBF, nact, bemap).astype(jnp.int32)

        ei = ei_scr[...]
        acc = jnp.zeros((N, 2), jnp.float32)
        for k in range(E):
            acc = acc + jnp.where(ei == k, po[0:1, k:k + 1], 0.0)
        pos_ref[...] = (acc + rk_scr[...]).astype(jnp.int32)


def _ffn_body(smap_ref, xs_ref, W1_ref, b1_ref, W2_ref, b2_ref, ys_ref, *, NBF):
    j = pl.program_id(0)
    nact = smap_ref[NBF]

    @pl.when(j < nact)
    def _():
        xb = xs_ref[...].astype(jnp.bfloat16)
        h1 = jax.lax.dot_general(xb, W1_ref[0], (((1,), (1,)), ((), ())),
                                 preferred_element_type=jnp.float32)
        h1 = _gelu(h1 + b1_ref[0]).astype(jnp.bfloat16)
        eo = jax.lax.dot_general(h1, W2_ref[0], (((1,), (1,)), ((), ())),
                                 preferred_element_type=jnp.float32)
        ys_ref[...] = eo + b2_ref[0]


def _make_scatter(N, D, NPAD, NW):
    rows_per_w = 2 * N // NW
    mesh = plsc.VectorSubcoreMesh(core_axis_name="c", subcore_axis_name="s")

    @functools.partial(
        pl.kernel, mesh=mesh,
        out_type=jax.ShapeDtypeStruct((NPAD, D), jnp.float32),
        scratch_types=[
            pltpu.VMEM((rows_per_w,), jnp.int32),
            pltpu.VMEM((rows_per_w, D), jnp.float32),
            pltpu.SemaphoreType.DMA,
        ],
    )
    def _scatter(xn_hbm, pos_hbm, xs_hbm, idx_v, rows_v, sem):
        wid = lax.axis_index("s") * 2 + lax.axis_index("c")
        abase = wid * rows_per_w
        tbase = abase % N
        pltpu.sync_copy(pos_hbm.at[pl.ds(abase, rows_per_w)], idx_v)
        pltpu.sync_copy(xn_hbm.at[pl.ds(tbase, rows_per_w)], rows_v)
        pltpu.async_copy(rows_v, xs_hbm.at[idx_v], sem).wait()

    return _scatter


def _make_combine(N, D, NW):
    CH = 32
    nch = N // (NW * CH)
    mesh = plsc.VectorSubcoreMesh(core_axis_name="c", subcore_axis_name="s")

    @functools.partial(
        pl.kernel, mesh=mesh,
        out_type=jax.ShapeDtypeStruct((N, D), jnp.float32),
        scratch_types=[
            pltpu.VMEM((CH, D), jnp.float32),
            pltpu.VMEM((CH, D), jnp.float32),
            pltpu.VMEM((CH, D), jnp.float32),
            pltpu.VMEM((CH,), jnp.int32),
            pltpu.VMEM((CH,), jnp.int32),
            pltpu.VMEM((CH * 16,), jnp.float32),
            pltpu.VMEM((CH * 16,), jnp.float32),
            pltpu.SemaphoreType.DMA,
            pltpu.SemaphoreType.DMA,
            pltpu.SemaphoreType.DMA,
        ],
    )
    def _combine(tok_hbm, ys_hbm, pos_hbm, g_hbm, out_hbm,
                 b0, b1, b2, i1v, i2v, g1v, g2v, s0, s1, s2):
        wid = lax.axis_index("s") * 2 + lax.axis_index("c")
        for c in range(nch):
            tb = wid * (nch * CH) + c * CH
            pltpu.sync_copy(pos_hbm.at[pl.ds(tb, CH)], i1v)
            pltpu.sync_copy(pos_hbm.at[pl.ds(N + tb, CH)], i2v)
            pltpu.sync_copy(g_hbm.at[pl.ds(tb * 16, CH * 16)], g1v)
            pltpu.sync_copy(g_hbm.at[pl.ds((N + tb) * 16, CH * 16)], g2v)
            cp0 = pltpu.async_copy(tok_hbm.at[pl.ds(tb, CH)], b0, s0)
            cp1 = pltpu.async_copy(ys_hbm.at[i1v], b1, s1)
            cp2 = pltpu.async_copy(ys_hbm.at[i2v], b2, s2)
            cp0.wait()
            cp1.wait()
            cp2.wait()

            def tokbody(t, _):
                g1 = g1v[pl.ds(t * 16, 16)]
                g2 = g2v[pl.ds(t * 16, 16)]

                def vbody(j, _):
                    sl = pl.ds(j * 16, 16)
                    b0[t, sl] = b0[t, sl] + g1 * b1[t, sl] + g2 * b2[t, sl]
                    return 0

                return lax.fori_loop(0, D // 16, vbody, 0)

            lax.fori_loop(0, CH, tokbody, 0)
            pltpu.sync_copy(b0, out_hbm.at[pl.ds(tb, CH)])

    return _combine


def kernel(tokens, alt_embedding, gamma, beta, Wg1, bg1, Wg2, bg2, W1, b1, W2, b2):
    B, N, D = tokens.shape
    ALT = alt_embedding.shape[-1]
    E, DFF, _ = W1.shape
    DG = Wg1.shape[0]
    TBA = 512
    NBA = N // TBA
    TBF = 256
    NBF = (2 * N) // TBF + (E - 1)
    NPAD = NBF * TBF
    NW = 32

    x = tokens.reshape(N, D)
    w1x = Wg1[:, :D]
    w1a = Wg1[:, D:]
    W1b = W1.astype(jnp.bfloat16)
    W2b = W2.astype(jnp.bfloat16)

    bodyA = functools.partial(_router_body, TBA=TBA, NBA=NBA, E=E, N=N,
                              TBF=TBF, NBF=NBF)
    full = lambda s: pl.BlockSpec(s, lambda b: (0,) * len(s))
    xn, gts, pos, smap32, lbv = pl.pallas_call(
        bodyA,
        grid=(NBA,),
        in_specs=[
            pl.BlockSpec((TBA, D), lambda b: (b, 0)),
            full((1, ALT)),
            full((1, D)),
            full((1, D)),
            full((DG, D)),
            full((DG, ALT)),
            full((1, DG)),
            full((E, DG)),
            full((1, E)),
        ],
        out_specs=[
            pl.BlockSpec((TBA, D), lambda b: (b, 0)),
            pl.BlockSpec((TBA, 2), lambda b: (b, 0)),
            full((N, 2)),
            full((1, 32)),
            full((1, 8)),
        ],
        out_shape=[
            jax.ShapeDtypeStruct((N, D), jnp.float32),
            jax.ShapeDtypeStruct((N, 2), jnp.float32),
            jax.ShapeDtypeStruct((N, 2), jnp.int32),
            jax.ShapeDtypeStruct((1, 32), jnp.int32),
            jax.ShapeDtypeStruct((1, 8), jnp.float32),
        ],
        scratch_shapes=[
            pltpu.VMEM((1, E), jnp.float32),
            pltpu.VMEM((2, E), jnp.float32),
            pltpu.VMEM((N, 2), jnp.int32),
            pltpu.VMEM((N, 2), jnp.float32),
        ],
        compiler_params=pltpu.CompilerParams(
            dimension_semantics=("arbitrary",),
        ),
    )(x, alt_embedding, gamma.reshape(1, D), beta.reshape(1, D),
      w1x, w1a, bg1.reshape(1, DG), Wg2, bg2.reshape(1, E))

    smap = smap32[0, :NBF + 1]
    posc = pos.T.reshape(2 * N)
    gc = jnp.repeat(gts.T.reshape(2 * N), 16)

    xs = _make_scatter(N, D, NPAD, NW)(xn, posc)

    bodyC = functools.partial(_ffn_body, NBF=NBF)
    ys = pl.pallas_call(
        bodyC,
        grid_spec=pltpu.PrefetchScalarGridSpec(
            num_scalar_prefetch=1,
            grid=(NBF,),
            in_specs=[
                pl.BlockSpec((TBF, D), lambda j, sm: (j, 0)),
                pl.BlockSpec((1, DFF, D), lambda j, sm: (sm[j], 0, 0)),
                pl.BlockSpec((1, 1, DFF), lambda j, sm: (sm[j], 0, 0)),
                pl.BlockSpec((1, D, DFF), lambda j, sm: (sm[j], 0, 0)),
                pl.BlockSpec((1, 1, D), lambda j, sm: (sm[j], 0, 0)),
            ],
            out_specs=pl.BlockSpec((TBF, D), lambda j, sm: (j, 0)),
        ),
        out_shape=jax.ShapeDtypeStruct((NPAD, D), jnp.float32),
        compiler_params=pltpu.CompilerParams(
            dimension_semantics=("arbitrary",),
        ),
    )(smap, xs, W1b, b1.reshape(E, 1, DFF), W2b, b2.reshape(E, 1, D))

    out = _make_combine(N, D, NW)(x, ys, posc, gc)
    return out.reshape(B, N, D), lbv[0, 0]


# speedup vs baseline: 1.2996x; 1.2567x over previous
import functools

import jax
import jax.numpy as jnp
from jax import lax
from jax.experimental import pallas as pl
from jax.experimental.pallas import tpu as pltpu
from jax.experimental.pallas import tpu_sc as plsc


def _gelu(x):
    return 0.5 * x * (1.0 + jax.lax.erf(x * 0.7071067811865476))


def _router_body(tok_ref, alt_ref, gamma_ref, beta_ref,
                 w1x_ref, w1a_ref, bg1_ref, wg2_ref, bg2_ref,
                 xn_ref, g_ref, pos_ref, smap_ref, lb_ref,
                 run_scr, fp_scr, ei_scr, rk_scr, *, TBA, NBA, E, N, TBF, NBF):
    b = pl.program_id(0)

    @pl.when(b == 0)
    def _init():
        run_scr[...] = jnp.zeros_like(run_scr)
        fp_scr[...] = jnp.zeros_like(fp_scr)

    tok = tok_ref[...]
    mu = jnp.mean(tok, axis=1, keepdims=True)
    xc = tok - mu
    var = jnp.mean(xc * xc, axis=1, keepdims=True)
    xn = xc * jax.lax.rsqrt(var + 1e-5) * gamma_ref[...] + beta_ref[...]
    xn_ref[...] = xn

    h = jax.lax.dot_general(xn, w1x_ref[...], (((1,), (1,)), ((), ())),
                            preferred_element_type=jnp.float32)
    ha = jax.lax.dot_general(alt_ref[...], w1a_ref[...], (((1,), (1,)), ((), ())),
                             preferred_element_type=jnp.float32)
    h = _gelu(h + ha + bg1_ref[...])
    logits = jax.lax.dot_general(h, wg2_ref[...], (((1,), (1,)), ((), ())),
                                 preferred_element_type=jnp.float32)
    logits = logits + bg2_ref[...]

    iota_e = jax.lax.broadcasted_iota(jnp.int32, logits.shape, 1)
    m1 = jnp.max(logits, axis=1, keepdims=True)
    i1 = jnp.min(jnp.where(logits == m1, iota_e, E), axis=1, keepdims=True)
    l2 = jnp.where(iota_e == i1, jnp.float32(-jnp.inf), logits)
    m2 = jnp.max(l2, axis=1, keepdims=True)
    i2 = jnp.min(jnp.where(l2 == m2, iota_e, E), axis=1, keepdims=True)
    r = jnp.exp(m2 - m1)
    den = 1.0 + r
    g_ref[...] = jnp.concatenate([1.0 / den, r / den], axis=1)

    a1 = (iota_e == i1).astype(jnp.float32)
    a2 = (iota_e == i2).astype(jnp.float32)
    ri = jax.lax.broadcasted_iota(jnp.int32, (TBA, TBA), 0)
    ci = jax.lax.broadcasted_iota(jnp.int32, (TBA, TBA), 1)
    tri = (ri > ci).astype(jnp.float32)
    cum1 = jax.lax.dot_general(tri, a1, (((1,), (0,)), ((), ())),
                               preferred_element_type=jnp.float32)
    cum2 = jax.lax.dot_general(tri, a2, (((1,), (0,)), ((), ())),
                               preferred_element_type=jnp.float32)
    base = run_scr[...]
    cs1 = jnp.sum(a1, axis=0, keepdims=True)
    cs2 = jnp.sum(a2, axis=0, keepdims=True)
    r1 = jnp.sum(a1 * (base + cum1), axis=1, keepdims=True)
    r2 = jnp.sum(a2 * (base + cs1 + cum2), axis=1, keepdims=True)
    ei_scr[pl.ds(b * TBA, TBA), :] = jnp.concatenate([i1, i2], axis=1)
    rk_scr[pl.ds(b * TBA, TBA), :] = jnp.concatenate([r1, r2], axis=1)
    run_scr[...] = base + cs1 + cs2

    p = jnp.exp(logits - m1)
    p = p / jnp.sum(p, axis=1, keepdims=True)
    fp_scr[0:1, :] += cs1
    fp_scr[1:2, :] += jnp.sum(p, axis=0, keepdims=True)

    @pl.when(b == NBA - 1)
    def _fin():
        lb = (E / (N * N)) * jnp.sum(fp_scr[0:1, :] * fp_scr[1:2, :])
        lb_ref[...] = jnp.full(lb_ref.shape, lb, jnp.float32)

        cnt = run_scr[...]
        pb_ = jnp.floor((cnt + (TBF - 1)) * (1.0 / TBF))
        li = jax.lax.broadcasted_iota(jnp.int32, (E, E), 0)
        lj = jax.lax.broadcasted_iota(jnp.int32, (E, E), 1)
        lower = (li <= lj).astype(jnp.float32)
        cum = jax.lax.dot_general(pb_, lower, (((1,), (0,)), ((), ())),
                                  preferred_element_type=jnp.float32)
        po = (cum - pb_) * TBF
        nact = cum[0:1, E - 1:E]
        jarr = jax.lax.broadcasted_iota(jnp.int32, smap_ref.shape, 1
                                        ).astype(jnp.float32)
        bemap = jnp.zeros(smap_ref.shape, jnp.float32)
        for k in range(E):
            bemap = bemap + (jarr >= cum[0:1, k:k + 1]).astype(jnp.float32)
        bemap = jnp.minimum(bemap, E - 1)
        smap_ref[...] = jnp.where(jarr == NBF, nact, bemap).astype(jnp.int32)

        ei = ei_scr[...]
        acc = jnp.zeros((N, 2), jnp.float32)
        for k in range(E):
            acc = acc + jnp.where(ei == k, po[0:1, k:k + 1], 0.0)
        pos_ref[...] = (acc + rk_scr[...]).astype(jnp.int32)


def _ffn_body(smap_ref, xs_ref, W1_ref, b1_ref, W2_ref, b2_ref, ys_ref, *, NBF):
    j = pl.program_id(0)
    nact = smap_ref[NBF]

    @pl.when(j < nact)
    def _():
        xb = xs_ref[...]
        h1 = jax.lax.dot_general(xb, W1_ref[0], (((1,), (1,)), ((), ())),
                                 preferred_element_type=jnp.float32)
        h1 = _gelu(h1 + b1_ref[0])
        eo = jax.lax.dot_general(h1, W2_ref[0], (((1,), (1,)), ((), ())),
                                 preferred_element_type=jnp.float32)
        ys_ref[...] = eo + b2_ref[0]


def _make_scatter(N, D, NPAD, NW):
    rows_per_w = 2 * N // NW
    mesh = plsc.VectorSubcoreMesh(core_axis_name="c", subcore_axis_name="s")

    @functools.partial(
        pl.kernel, mesh=mesh,
        out_type=jax.ShapeDtypeStruct((NPAD, D), jnp.float32),
        scratch_types=[
            pltpu.VMEM((rows_per_w,), jnp.int32),
            pltpu.VMEM((rows_per_w, D), jnp.float32),
            pltpu.SemaphoreType.DMA,
        ],
    )
    def _scatter(xn_hbm, pos_hbm, xs_hbm, idx_v, rows_v, sem):
        wid = lax.axis_index("s") * 2 + lax.axis_index("c")
        abase = wid * rows_per_w
        tbase = abase % N
        pltpu.sync_copy(pos_hbm.at[pl.ds(abase, rows_per_w)], idx_v)
        pltpu.sync_copy(xn_hbm.at[pl.ds(tbase, rows_per_w)], rows_v)
        pltpu.async_copy(rows_v, xs_hbm.at[idx_v], sem).wait()

    return _scatter


def _make_combine(N, D, NW):
    CH = 32
    nch = N // (NW * CH)
    mesh = plsc.VectorSubcoreMesh(core_axis_name="c", subcore_axis_name="s")

    @functools.partial(
        pl.kernel, mesh=mesh,
        out_type=jax.ShapeDtypeStruct((N, D), jnp.float32),
        scratch_types=[
            pltpu.VMEM((CH, D), jnp.float32),
            pltpu.VMEM((CH, D), jnp.float32),
            pltpu.VMEM((CH, D), jnp.float32),
            pltpu.VMEM((CH,), jnp.int32),
            pltpu.VMEM((CH,), jnp.int32),
            pltpu.VMEM((CH * 16,), jnp.float32),
            pltpu.VMEM((CH * 16,), jnp.float32),
            pltpu.SemaphoreType.DMA,
            pltpu.SemaphoreType.DMA,
            pltpu.SemaphoreType.DMA,
        ],
    )
    def _combine(tok_hbm, ys_hbm, pos_hbm, g_hbm, out_hbm,
                 b0, b1, b2, i1v, i2v, g1v, g2v, s0, s1, s2):
        wid = lax.axis_index("s") * 2 + lax.axis_index("c")
        for c in range(nch):
            tb = wid * (nch * CH) + c * CH
            pltpu.sync_copy(pos_hbm.at[pl.ds(tb, CH)], i1v)
            pltpu.sync_copy(pos_hbm.at[pl.ds(N + tb, CH)], i2v)
            pltpu.sync_copy(g_hbm.at[pl.ds(tb * 16, CH * 16)], g1v)
            pltpu.sync_copy(g_hbm.at[pl.ds((N + tb) * 16, CH * 16)], g2v)
            cp0 = pltpu.async_copy(tok_hbm.at[pl.ds(tb, CH)], b0, s0)
            cp1 = pltpu.async_copy(ys_hbm.at[i1v], b1, s1)
            cp2 = pltpu.async_copy(ys_hbm.at[i2v], b2, s2)
            cp0.wait()
            cp1.wait()
            cp2.wait()

            def tokbody(t, _):
                g1 = g1v[pl.ds(t * 16, 16)]
                g2 = g2v[pl.ds(t * 16, 16)]

                def vbody(j, _):
                    sl = pl.ds(j * 16, 16)
                    b0[t, sl] = b0[t, sl] + g1 * b1[t, sl] + g2 * b2[t, sl]
                    return 0

                return lax.fori_loop(0, D // 16, vbody, 0)

            lax.fori_loop(0, CH, tokbody, 0)
            pltpu.sync_copy(b0, out_hbm.at[pl.ds(tb, CH)])

    return _combine


def kernel(tokens, alt_embedding, gamma, beta, Wg1, bg1, Wg2, bg2, W1, b1, W2, b2):
    B, N, D = tokens.shape
    ALT = alt_embedding.shape[-1]
    E, DFF, _ = W1.shape
    DG = Wg1.shape[0]
    TBA = 512
    NBA = N // TBA
    TBF = 256
    NBF = (2 * N) // TBF + (E - 1)
    NPAD = NBF * TBF
    NW = 32

    x = tokens.reshape(N, D)
    w1x = Wg1[:, :D]
    w1a = Wg1[:, D:]

    bodyA = functools.partial(_router_body, TBA=TBA, NBA=NBA, E=E, N=N,
                              TBF=TBF, NBF=NBF)
    full = lambda s: pl.BlockSpec(s, lambda b: (0,) * len(s))
    xn, gts, pos, smap32, lbv = pl.pallas_call(
        bodyA,
        grid=(NBA,),
        in_specs=[
            pl.BlockSpec((TBA, D), lambda b: (b, 0)),
            full((1, ALT)),
            full((1, D)),
            full((1, D)),
            full((DG, D)),
            full((DG, ALT)),
            full((1, DG)),
            full((E, DG)),
            full((1, E)),
        ],
        out_specs=[
            pl.BlockSpec((TBA, D), lambda b: (b, 0)),
            pl.BlockSpec((TBA, 2), lambda b: (b, 0)),
            full((N, 2)),
            full((1, 32)),
            full((1, 8)),
        ],
        out_shape=[
            jax.ShapeDtypeStruct((N, D), jnp.float32),
            jax.ShapeDtypeStruct((N, 2), jnp.float32),
            jax.ShapeDtypeStruct((N, 2), jnp.int32),
            jax.ShapeDtypeStruct((1, 32), jnp.int32),
            jax.ShapeDtypeStruct((1, 8), jnp.float32),
        ],
        scratch_shapes=[
            pltpu.VMEM((1, E), jnp.float32),
            pltpu.VMEM((2, E), jnp.float32),
            pltpu.VMEM((N, 2), jnp.int32),
            pltpu.VMEM((N, 2), jnp.float32),
        ],
        compiler_params=pltpu.CompilerParams(
            dimension_semantics=("arbitrary",),
        ),
    )(x, alt_embedding, gamma.reshape(1, D), beta.reshape(1, D),
      w1x, w1a, bg1.reshape(1, DG), Wg2, bg2.reshape(1, E))

    smap = smap32[0, :NBF + 1]
    posc = pos.T.reshape(2 * N)
    gc = jnp.repeat(gts.T.reshape(2 * N), 16)

    xs = _make_scatter(N, D, NPAD, NW)(xn, posc)

    bodyC = functools.partial(_ffn_body, NBF=NBF)
    ys = pl.pallas_call(
        bodyC,
        grid_spec=pltpu.PrefetchScalarGridSpec(
            num_scalar_prefetch=1,
            grid=(NBF,),
            in_specs=[
                pl.BlockSpec((TBF, D), lambda j, sm: (j, 0)),
                pl.BlockSpec((1, DFF, D), lambda j, sm: (sm[j], 0, 0)),
                pl.BlockSpec((1, 1, DFF), lambda j, sm: (sm[j], 0, 0)),
                pl.BlockSpec((1, D, DFF), lambda j, sm: (sm[j], 0, 0)),
                pl.BlockSpec((1, 1, D), lambda j, sm: (sm[j], 0, 0)),
            ],
            out_specs=pl.BlockSpec((TBF, D), lambda j, sm: (j, 0)),
        ),
        out_shape=jax.ShapeDtypeStruct((NPAD, D), jnp.float32),
        compiler_params=pltpu.CompilerParams(
            dimension_semantics=("arbitrary",),
        ),
    )(smap, xs, W1, b1.reshape(E, 1, DFF), W2, b2.reshape(E, 1, D))

    out = _make_combine(N, D, NW)(x, ys, posc, gc)
    return out.reshape(B, N, D), lbv[0, 0]
